# baseline (speedup 1.0000x reference)
import functools

import jax
import jax.numpy as jnp
from jax import lax
from jax.experimental import pallas as pl
from jax.experimental.pallas import tpu as pltpu
from jax.experimental.pallas import tpu_sc as plsc

B, N, D, M = 16, 1024, 768, 300
MP = 304
PW0 = 160
PW1 = 136
MT = 296
LANES = 16


def _sc_gather(seg_flat, idx_flat):
    mesh = plsc.VectorSubcoreMesh(core_axis_name="c", subcore_axis_name="s")

    @functools.partial(
        pl.kernel,
        mesh=mesh,
        out_type=jax.ShapeDtypeStruct((B, M, D), jnp.float32),
        scratch_types=[
            pltpu.VMEM((PW0,), jnp.int32),
            pltpu.VMEM((PW0, D), jnp.float32),
            pltpu.SemaphoreType.DMA,
        ],
    )
    def k(seg_hbm, idx_hbm, out_hbm, idx_v, rows_v, sem):
        wid = lax.axis_index("s") * 2 + lax.axis_index("c")
        b = wid // 2
        half = wid % 2
        row_off = b * N

        @pl.when(half == 0)
        def _():
            pltpu.sync_copy(idx_hbm.at[pl.ds(b * MP, PW0)], idx_v)
            for j in range(PW0 // LANES):
                sl = pl.ds(j * LANES, LANES)
                idx_v[sl] = idx_v[sl] + row_off
            for c in range(2):
                pltpu.async_copy(
                    seg_hbm.at[idx_v.at[pl.ds(c * 80, 80)]],
                    rows_v.at[pl.ds(c * 80, 80)],
                    sem,
                ).wait()
            pltpu.sync_copy(rows_v, out_hbm.at[b, pl.ds(0, PW0), :])

        @pl.when(half == 1)
        def _():
            pltpu.sync_copy(
                idx_hbm.at[pl.ds(b * MP + PW0, 144)], idx_v.at[pl.ds(0, 144)]
            )
            for j in range(144 // LANES):
                sl = pl.ds(j * LANES, LANES)
                idx_v[sl] = idx_v[sl] + row_off
            pltpu.async_copy(
                seg_hbm.at[idx_v.at[pl.ds(0, 72)]],
                rows_v.at[pl.ds(0, 72)],
                sem,
            ).wait()
            pltpu.async_copy(
                seg_hbm.at[idx_v.at[pl.ds(72, 64)]],
                rows_v.at[pl.ds(72, 64)],
                sem,
            ).wait()
            pltpu.sync_copy(
                rows_v.at[pl.ds(0, PW1)], out_hbm.at[b, pl.ds(PW0, PW1), :]
            )

    return k(seg_flat, idx_flat)


def _tail_patch(out_sc, seg_out, tail_idx):
    ntail = M - MT

    def body(idx_ref, outsc_ref, seg_ref, out_ref):
        b = pl.program_id(0)
        i = pl.program_id(1)
        r = idx_ref[b, i] % 8
        out_ref[0, i, :] = seg_ref[0, r, :]

    grid_spec = pltpu.PrefetchScalarGridSpec(
        num_scalar_prefetch=1,
        grid=(B, ntail),
        in_specs=[
            pl.BlockSpec(memory_space=pltpu.MemorySpace.HBM),
            pl.BlockSpec((1, 8, D), lambda b, i, idx: (b, idx[b, i] // 8, 0)),
        ],
        out_specs=pl.BlockSpec((1, 8, D), lambda b, i, idx: (b, MT // 8, 0)),
    )
    return pl.pallas_call(
        body,
        grid_spec=grid_spec,
        out_shape=jax.ShapeDtypeStruct((B, M, D), jnp.float32),
        input_output_aliases={1: 0},
        compiler_params=pltpu.CompilerParams(
            dimension_semantics=("arbitrary", "arbitrary"),
        ),
    )(tail_idx, out_sc, seg_out)


def kernel(seg_out, det_out, det_scores, det_indices):
    idx = det_indices.astype(jnp.int32)
    idx_padded = jnp.pad(idx, ((0, 0), (0, MP - M)))
    out = _sc_gather(seg_out.reshape(B * N, D), idx_padded.reshape(B * MP))
    return _tail_patch(out, seg_out, idx[:, MT:M])


# speedup vs baseline: 1.4577x; 1.4577x over previous
import functools

import jax
import jax.numpy as jnp
from jax import lax
from jax.experimental import pallas as pl
from jax.experimental.pallas import tpu as pltpu
from jax.experimental.pallas import tpu_sc as plsc

B, N, D, M = 16, 1024, 768, 300
MP = 320
PW = 160
CH = 80
LANES = 16


def _sc_gather(seg_flat, idx_flat):
    mesh = plsc.VectorSubcoreMesh(core_axis_name="c", subcore_axis_name="s")

    @functools.partial(
        pl.kernel,
        mesh=mesh,
        out_type=jax.ShapeDtypeStruct((B, M, D), jnp.float32),
        scratch_types=[
            pltpu.VMEM((PW,), jnp.int32),
            pltpu.VMEM((2, CH), jnp.int32),
            pltpu.VMEM((PW, D), jnp.float32),
            pltpu.SemaphoreType.DMA,
        ],
    )
    def k(seg_hbm, idx_hbm, out_hbm, idx_v, didx_v, rows_v, sem):
        wid = lax.axis_index("s") * 2 + lax.axis_index("c")
        b = wid // 2
        half = wid % 2
        base = b * MP + half * PW
        row_off = b * N
        dbase = half * PW

        pltpu.sync_copy(idx_hbm.at[pl.ds(base, PW)], idx_v)
        iot = lax.iota(jnp.int32, 16)
        for j in range(PW // LANES):
            sl = pl.ds(j * LANES, LANES)
            idx_v[sl] = idx_v[sl] + row_off
        for c in range(PW // CH):
            for j in range(CH // LANES):
                didx_v[c, pl.ds(j * LANES, LANES)] = jnp.minimum(
                    dbase + c * CH + j * LANES + iot, M - 1
                )
        for c in range(PW // CH):
            pltpu.async_copy(
                seg_hbm.at[idx_v.at[pl.ds(c * CH, CH)]],
                rows_v.at[pl.ds(c * CH, CH)],
                sem,
            ).wait()
        for c in range(PW // CH):
            pltpu.async_copy(
                rows_v.at[pl.ds(c * CH, CH)],
                out_hbm.at[b].at[didx_v.at[c]],
                sem,
            ).wait()

    return k(seg_flat, idx_flat)


def kernel(seg_out, det_out, det_scores, det_indices):
    idx = det_indices.astype(jnp.int32)
    idx = jnp.pad(idx, ((0, 0), (0, MP - M)), mode="edge")
    return _sc_gather(seg_out.reshape(B * N, D), idx.reshape(B * MP))


# speedup vs baseline: 1.5590x; 1.0695x over previous
import functools

import jax
import jax.numpy as jnp
from jax import lax
from jax.experimental import pallas as pl
from jax.experimental.pallas import tpu as pltpu
from jax.experimental.pallas import tpu_sc as plsc

B, N, D, M = 16, 1024, 768, 300
MP = 304
PW0 = 160
PW1 = 144
LANES = 16


def _sc_gather(seg_flat, idx_flat):
    mesh = plsc.VectorSubcoreMesh(core_axis_name="c", subcore_axis_name="s")

    @functools.partial(
        pl.kernel,
        mesh=mesh,
        out_type=jax.ShapeDtypeStruct((B, M, D), jnp.float32),
        scratch_types=[
            pltpu.VMEM((PW0,), jnp.int32),
            pltpu.VMEM((LANES,), jnp.int32),
            pltpu.VMEM((PW0, D), jnp.float32),
            pltpu.SemaphoreType.DMA,
        ],
    )
    def k(seg_hbm, idx_hbm, out_hbm, idx_v, didx_v, rows_v, sem):
        wid = lax.axis_index("s") * 2 + lax.axis_index("c")
        b = wid // 2
        half = wid % 2
        row_off = b * N
        iot = lax.iota(jnp.int32, 16)

        @pl.when(half == 0)
        def _():
            pltpu.sync_copy(idx_hbm.at[pl.ds(b * MP, PW0)], idx_v)
            for j in range(PW0 // LANES):
                sl = pl.ds(j * LANES, LANES)
                idx_v[sl] = idx_v[sl] + row_off
            for c in range(2):
                pltpu.async_copy(
                    seg_hbm.at[idx_v.at[pl.ds(c * 80, 80)]],
                    rows_v.at[pl.ds(c * 80, 80)],
                    sem,
                ).wait()
            pltpu.sync_copy(rows_v, out_hbm.at[b, pl.ds(0, PW0), :])

        @pl.when(half == 1)
        def _():
            pltpu.sync_copy(
                idx_hbm.at[pl.ds(b * MP + PW0, PW1)], idx_v.at[pl.ds(0, PW1)]
            )
            for j in range(PW1 // LANES):
                sl = pl.ds(j * LANES, LANES)
                idx_v[sl] = idx_v[sl] + row_off
            didx_v[...] = jnp.minimum(288 + iot, M - 1)
            pltpu.async_copy(
                seg_hbm.at[idx_v.at[pl.ds(0, 80)]],
                rows_v.at[pl.ds(0, 80)],
                sem,
            ).wait()
            pltpu.async_copy(
                seg_hbm.at[idx_v.at[pl.ds(80, 64)]],
                rows_v.at[pl.ds(80, 64)],
                sem,
            ).wait()
            pltpu.sync_copy(
                rows_v.at[pl.ds(0, 136)], out_hbm.at[b, pl.ds(PW0, 136), :]
            )
            pltpu.async_copy(
                rows_v.at[pl.ds(128, LANES)],
                out_hbm.at[b].at[didx_v],
                sem,
            ).wait()

    return k(seg_flat, idx_flat)


def kernel(seg_out, det_out, det_scores, det_indices):
    idx = det_indices.astype(jnp.int32)
    idx = jnp.pad(idx, ((0, 0), (0, MP - M)), mode="edge")
    return _sc_gather(seg_out.reshape(B * N, D), idx.reshape(B * MP))
